# baseline (speedup 1.0000x reference)
import jax
import jax.numpy as jnp
from jax import lax
from jax.experimental import pallas as pl
from jax.experimental.pallas import tpu as pltpu
from jax.experimental.pallas import tpu_sc as plsc

BATCH = 4096
SEQ_LEN = 200
EMBED_DIM = 64
VOCAB = 1000000

NUM_CORES = 2
NUM_SUBCORES = 16
NUM_WORKERS = NUM_CORES * NUM_SUBCORES

BLOCK_B = BATCH // NUM_WORKERS
NBUF = 4
LANES = 16
GROUPS = EMBED_DIM // LANES

TILE_C = 128
NUM_TCOLS = (VOCAB + TILE_C - 1) // TILE_C
FULL_TCOLS = VOCAB // TILE_C
LAST_VALID = VOCAB - FULL_TCOLS * TILE_C
COLS_PER_W = (NUM_TCOLS + NUM_WORKERS - 1) // NUM_WORKERS


def _wid():
    return lax.axis_index("s") * NUM_CORES + lax.axis_index("c")


def _repack_body(tab_t_hbm, flat_hbm, stage_bufs, trans_bufs, gsems, wsems):
    wid = _wid()
    iota = lax.iota(jnp.int32, LANES)
    sc_idx = [iota * EMBED_DIM + tg * LANES * EMBED_DIM for tg in range(8)]

    def col(k):
        return wid + k * NUM_WORKERS

    def read_copy(b, k):
        return pltpu.make_async_copy(
            tab_t_hbm.at[:, pl.ds(col(k) * TILE_C, TILE_C)], stage_bufs[b],
            gsems[b])

    def write_copy(b, k):
        return pltpu.make_async_copy(
            trans_bufs[b], flat_hbm.at[pl.ds(col(k) * TILE_C * EMBED_DIM,
                                             TILE_C * EMBED_DIM)], wsems[b])

    def write_copy_tail(b, k):
        return pltpu.make_async_copy(
            trans_bufs[b].at[pl.ds(0, LAST_VALID * EMBED_DIM)],
            flat_hbm.at[pl.ds(col(k) * TILE_C * EMBED_DIM,
                              LAST_VALID * EMBED_DIM)], wsems[b])

    def transpose(b):
        def ebody(e, carry):
            base = jnp.broadcast_to(e, (LANES,)).astype(jnp.int32)
            for tg in range(8):
                v = stage_bufs[b][e, pl.ds(tg * LANES, LANES)]
                plsc.store_scatter(trans_bufs[b], [sc_idx[tg] + base], v)
            return carry

        lax.fori_loop(0, EMBED_DIM, ebody, 0, unroll=4)

    for b in range(NBUF - 1):
        @pl.when(col(b) < NUM_TCOLS)
        def _():
            read_copy(b, b).start()

    def outer(o, carry):
        for b in range(NBUF):
            k = o * NBUF + b
            c = col(k)

            @pl.when(c < NUM_TCOLS)
            def _():
                read_copy(b, k).wait()

                @pl.when(k >= NBUF)
                def _():
                    write_copy(b, k - NBUF).wait()

                transpose(b)

            @pl.when(col(k + NBUF - 1) < NUM_TCOLS)
            def _():
                read_copy((b - 1) % NBUF, k + NBUF - 1).start()

            @pl.when(c < FULL_TCOLS)
            def _():
                write_copy(b, k).start()

            @pl.when(c == FULL_TCOLS)
            def _():
                write_copy_tail(b, k).start()
        return carry

    n_outer = COLS_PER_W // NBUF
    lax.fori_loop(0, n_outer, outer, 0, unroll=False)

    for k in range(n_outer * NBUF, COLS_PER_W):
        b = k % NBUF
        write_copy(b, k - NBUF).wait()

        @pl.when(col(k) < NUM_TCOLS)
        def _():
            read_copy(b, k).wait()
            transpose(b)

        @pl.when(col(k) < FULL_TCOLS)
        def _():
            write_copy(b, k).start()

        @pl.when(col(k) == FULL_TCOLS)
        def _():
            write_copy_tail(b, k).start()

    for k in range(COLS_PER_W - NBUF, COLS_PER_W):
        b = k % NBUF
        if k < n_outer * NBUF:
            write_copy(b, k).wait()
        else:
            @pl.when(col(k) < FULL_TCOLS)
            def _():
                write_copy(b, k).wait()

            @pl.when(col(k) == FULL_TCOLS)
            def _():
                write_copy_tail(b, k).wait()


def _embed_body(idx_hbm, table_hbm, pos_hbm, out_hbm, idx_v, rows_v, out_v,
                pos_v, gsems, wsems):
    wid = _wid()
    b0 = wid * BLOCK_B

    pltpu.sync_copy(pos_hbm, pos_v)
    pltpu.sync_copy(idx_hbm.at[:, pl.ds(b0, BLOCK_B)], idx_v)

    iota = lax.iota(jnp.int32, LANES)
    row_idx = [iota + g * LANES for g in range(GROUPS)]

    def gather_copy(b, l):
        return pltpu.make_async_copy(
            table_hbm.at[idx_v.at[l]], rows_v.at[b], gsems[b])

    def write_copy(b, l):
        return pltpu.make_async_copy(
            out_v.at[b], out_hbm.at[l, :, pl.ds(b0, BLOCK_B)], wsems[b])

    def transpose_add(b, l):
        pos_g = [pos_v[l, pl.ds(g * LANES, LANES)] for g in range(GROUPS)]

        def bbody(bb, carry):
            col_idx = jnp.broadcast_to(bb, (LANES,)).astype(jnp.int32)
            for g in range(GROUPS):
                v = rows_v[b, bb, pl.ds(g * LANES, LANES)] + pos_g[g]
                plsc.store_scatter(out_v.at[b], [row_idx[g], col_idx], v)
            return carry

        lax.fori_loop(0, BLOCK_B, bbody, 0, unroll=4)

    for b in range(NBUF - 1):
        gather_copy(b, b).start()

    def outer(o, carry):
        for b in range(NBUF):
            l = o * NBUF + b
            gather_copy(b, l).wait()

            @pl.when(l >= NBUF)
            def _():
                write_copy(b, l - NBUF).wait()

            transpose_add(b, l)

            @pl.when(l + NBUF - 1 <= SEQ_LEN - 1)
            def _():
                gather_copy((b - 1) % NBUF, l + NBUF - 1).start()

            write_copy(b, l).start()
        return carry

    lax.fori_loop(0, SEQ_LEN // NBUF, outer, 0, unroll=False)

    for b in range(NBUF):
        write_copy(b, SEQ_LEN - NBUF + b).wait()


@jax.jit
def _embed(inputs, token_table, pos_table):
    mesh = plsc.VectorSubcoreMesh(
        core_axis_name="c", subcore_axis_name="s", num_cores=NUM_CORES,
        num_subcores=NUM_SUBCORES)

    repack = pl.kernel(
        _repack_body,
        out_type=jax.ShapeDtypeStruct((VOCAB * EMBED_DIM,), jnp.float32),
        mesh=mesh,
        scratch_types=[
            [pltpu.VMEM((EMBED_DIM, TILE_C), jnp.float32)] * NBUF,
            [pltpu.VMEM((TILE_C * EMBED_DIM,), jnp.float32)] * NBUF,
            [pltpu.SemaphoreType.DMA] * NBUF,
            [pltpu.SemaphoreType.DMA] * NBUF,
        ],
        compiler_params=pltpu.CompilerParams(use_tc_tiling_on_sc=True,
                                             needs_layout_passes=False),
    )
    table_flat = repack(jnp.transpose(token_table))
    table_rm = table_flat.reshape(VOCAB, EMBED_DIM)

    idx_t = jnp.transpose(inputs)
    f = pl.kernel(
        _embed_body,
        out_type=jax.ShapeDtypeStruct((SEQ_LEN, EMBED_DIM, BATCH),
                                      jnp.float32),
        mesh=mesh,
        scratch_types=[
            pltpu.VMEM((SEQ_LEN, BLOCK_B), jnp.int32),
            pltpu.VMEM((NBUF, BLOCK_B, EMBED_DIM), jnp.float32),
            pltpu.VMEM((NBUF, EMBED_DIM, BLOCK_B), jnp.float32),
            pltpu.VMEM((SEQ_LEN, EMBED_DIM), jnp.float32),
            [pltpu.SemaphoreType.DMA] * NBUF,
            [pltpu.SemaphoreType.DMA] * NBUF,
        ],
        compiler_params=pltpu.CompilerParams(use_tc_tiling_on_sc=False,
                                             needs_layout_passes=False),
    )
    out_t = f(idx_t, table_rm, pos_table)
    return jnp.transpose(out_t, (2, 0, 1))


def kernel(inputs, token_table, pos_table):
    return _embed(inputs, token_table, pos_table)


# speedup vs baseline: 1.2883x; 1.2883x over previous
import jax
import jax.numpy as jnp
from jax import lax
from jax.experimental import pallas as pl
from jax.experimental.pallas import tpu as pltpu
from jax.experimental.pallas import tpu_sc as plsc

BATCH = 4096
SEQ_LEN = 200
EMBED_DIM = 64
VOCAB = 1000000

NUM_CORES = 2
NUM_SUBCORES = 16
NUM_WORKERS = NUM_CORES * NUM_SUBCORES

BLOCK_B = BATCH // NUM_WORKERS
NBUF = 4
LANES = 16
GROUPS = EMBED_DIM // LANES

TILE_C = 128
NUM_TCOLS = (VOCAB + TILE_C - 1) // TILE_C
COLS_PER_W = (NUM_TCOLS + NUM_WORKERS - 1) // NUM_WORKERS

TPAD = EMBED_DIM
BPAD = BLOCK_B + 1
TROWS = NUM_TCOLS * TILE_C


def _wid():
    return lax.axis_index("s") * NUM_CORES + lax.axis_index("c")


def _repack_body(tab_t_hbm, flat_hbm, stage_bufs, trans_bufs, gsems, wsems):
    wid = _wid()
    iota = lax.iota(jnp.int32, LANES)
    tok_base = [(iota + tg * LANES) * TPAD for tg in range(8)]

    def col(k):
        return wid + k * NUM_WORKERS

    def read_copy(b, k):
        return pltpu.make_async_copy(
            tab_t_hbm.at[:, pl.ds(col(k) * TILE_C, TILE_C)], stage_bufs[b],
            gsems[b])

    def write_copy(b, k):
        return pltpu.make_async_copy(
            trans_bufs[b],
            flat_hbm.at[pl.ds(col(k) * (TILE_C * TPAD), TILE_C * TPAD)],
            wsems[b])

    def transpose(b):
        def ebody(e, carry):
            ev = jnp.broadcast_to(e, (LANES,)).astype(jnp.int32)
            for tg in range(8):
                v = stage_bufs[b][e, pl.ds(tg * LANES, LANES)]
                plsc.store_scatter(trans_bufs[b], [tok_base[tg] + ev], v)
            return carry

        lax.fori_loop(0, EMBED_DIM, ebody, 0, unroll=4)

    for b in range(NBUF - 1):
        @pl.when(col(b) < NUM_TCOLS)
        def _():
            read_copy(b, b).start()

    def outer(o, carry):
        for b in range(NBUF):
            k = o * NBUF + b
            c = col(k)

            @pl.when(c < NUM_TCOLS)
            def _():
                read_copy(b, k).wait()

                @pl.when(k >= NBUF)
                def _():
                    write_copy(b, k - NBUF).wait()

                transpose(b)

            @pl.when(col(k + NBUF - 1) < NUM_TCOLS)
            def _():
                read_copy((b - 1) % NBUF, k + NBUF - 1).start()

            @pl.when(c < NUM_TCOLS)
            def _():
                write_copy(b, k).start()
        return carry

    n_outer = COLS_PER_W // NBUF
    lax.fori_loop(0, n_outer, outer, 0, unroll=False)

    for k in range(n_outer * NBUF, COLS_PER_W):
        b = k % NBUF
        write_copy(b, k - NBUF).wait()

        @pl.when(col(k) < NUM_TCOLS)
        def _():
            read_copy(b, k).wait()
            transpose(b)
            write_copy(b, k).start()

    for k in range(COLS_PER_W - NBUF, COLS_PER_W):
        b = k % NBUF
        if k < n_outer * NBUF:
            write_copy(b, k).wait()
        else:
            @pl.when(col(k) < NUM_TCOLS)
            def _():
                write_copy(b, k).wait()


def _embed_body(idx_hbm, table_hbm, pos_hbm, out_hbm, idx_v, rows_v, out_v,
                pos_v, gsems, wsems):
    wid = _wid()
    b0 = wid * BLOCK_B

    pltpu.sync_copy(pos_hbm, pos_v)
    pltpu.sync_copy(idx_hbm.at[:, pl.ds(b0, BLOCK_B)], idx_v)

    iota = lax.iota(jnp.int32, LANES)
    row_idx = [iota + g * LANES for g in range(GROUPS)]

    def gather_copy(b, l):
        return pltpu.make_async_copy(
            table_hbm.at[idx_v.at[l]], rows_v.at[b], gsems[b])

    def write_copy(b, l):
        return pltpu.make_async_copy(
            out_v.at[b, :, pl.ds(0, BLOCK_B)],
            out_hbm.at[l, :, pl.ds(b0, BLOCK_B)], wsems[b])

    def transpose_add(b, l):
        pos_g = [pos_v[l, pl.ds(g * LANES, LANES)] for g in range(GROUPS)]

        def bbody(bb, carry):
            col_idx = jnp.broadcast_to(bb, (LANES,)).astype(jnp.int32)
            for g in range(GROUPS):
                v = rows_v[b, bb, pl.ds(g * LANES, LANES)] + pos_g[g]
                plsc.store_scatter(out_v.at[b], [row_idx[g], col_idx], v)
            return carry

        lax.fori_loop(0, BLOCK_B, bbody, 0, unroll=4)

    for b in range(NBUF - 1):
        gather_copy(b, b).start()

    def outer(o, carry):
        for b in range(NBUF):
            l = o * NBUF + b
            gather_copy(b, l).wait()

            @pl.when(l >= NBUF)
            def _():
                write_copy(b, l - NBUF).wait()

            transpose_add(b, l)

            @pl.when(l + NBUF - 1 <= SEQ_LEN - 1)
            def _():
                gather_copy((b - 1) % NBUF, l + NBUF - 1).start()

            write_copy(b, l).start()
        return carry

    lax.fori_loop(0, SEQ_LEN // NBUF, outer, 0, unroll=False)

    for b in range(NBUF):
        write_copy(b, SEQ_LEN - NBUF + b).wait()


@jax.jit
def _embed(inputs, token_table, pos_table):
    mesh = plsc.VectorSubcoreMesh(
        core_axis_name="c", subcore_axis_name="s", num_cores=NUM_CORES,
        num_subcores=NUM_SUBCORES)

    repack = pl.kernel(
        _repack_body,
        out_type=jax.ShapeDtypeStruct((TROWS * TPAD,), jnp.float32),
        mesh=mesh,
        scratch_types=[
            [pltpu.VMEM((EMBED_DIM, TILE_C), jnp.float32)] * NBUF,
            [pltpu.VMEM((TILE_C * TPAD,), jnp.float32)] * NBUF,
            [pltpu.SemaphoreType.DMA] * NBUF,
            [pltpu.SemaphoreType.DMA] * NBUF,
        ],
        compiler_params=pltpu.CompilerParams(use_tc_tiling_on_sc=True,
                                             needs_layout_passes=False),
    )
    table_flat = repack(jnp.transpose(token_table))
    table_rm = table_flat.reshape(TROWS, TPAD)

    idx_t = jnp.transpose(inputs)
    f = pl.kernel(
        _embed_body,
        out_type=jax.ShapeDtypeStruct((SEQ_LEN, EMBED_DIM, BATCH),
                                      jnp.float32),
        mesh=mesh,
        scratch_types=[
            pltpu.VMEM((SEQ_LEN, BLOCK_B), jnp.int32),
            pltpu.VMEM((NBUF, BLOCK_B, TPAD), jnp.float32),
            pltpu.VMEM((NBUF, EMBED_DIM, BPAD), jnp.float32),
            pltpu.VMEM((SEQ_LEN, EMBED_DIM), jnp.float32),
            [pltpu.SemaphoreType.DMA] * NBUF,
            [pltpu.SemaphoreType.DMA] * NBUF,
        ],
        compiler_params=pltpu.CompilerParams(use_tc_tiling_on_sc=False,
                                             needs_layout_passes=False),
    )
    out_t = f(idx_t, table_rm, pos_table)
    return jnp.transpose(out_t, (2, 0, 1))


def kernel(inputs, token_table, pos_table):
    return _embed(inputs, token_table, pos_table)


# speedup vs baseline: 1.6865x; 1.3090x over previous
import jax
import jax.numpy as jnp
from jax import lax
from jax.experimental import pallas as pl
from jax.experimental.pallas import tpu as pltpu
from jax.experimental.pallas import tpu_sc as plsc

BATCH = 4096
SEQ_LEN = 200
EMBED_DIM = 64
VOCAB = 1000000

NUM_CORES = 2
NUM_SUBCORES = 16
NUM_WORKERS = NUM_CORES * NUM_SUBCORES

BLOCK_B = BATCH // NUM_WORKERS
NBUF = 4
LANES = 16
GROUPS = EMBED_DIM // LANES

TILE_C = 128
NUM_TCOLS = (VOCAB + TILE_C - 1) // TILE_C
COLS_PER_W = (NUM_TCOLS + NUM_WORKERS - 1) // NUM_WORKERS

TPAD = EMBED_DIM
BPAD = BLOCK_B + 1
TROWS = NUM_TCOLS * TILE_C


def _wid():
    return lax.axis_index("s") * NUM_CORES + lax.axis_index("c")


def _repack_body(tab_t_hbm, flat_hbm, stage_bufs, trans_bufs, gsems, wsems):
    wid = _wid()
    iota = lax.iota(jnp.int32, LANES)
    rot = [jnp.bitwise_and(iota + k, LANES - 1) for k in range(LANES)]
    rot64 = [r * EMBED_DIM for r in rot]
    ge = [iota + be for be in range(0, EMBED_DIM, LANES)]

    def col(k):
        return wid + k * NUM_WORKERS

    def read_copy(b, k):
        return pltpu.make_async_copy(
            tab_t_hbm.at[:, pl.ds(col(k) * TILE_C, TILE_C)], stage_bufs[b],
            gsems[b])

    def write_copy(b, k):
        return pltpu.make_async_copy(
            trans_bufs[b],
            flat_hbm.at[pl.ds(col(k) * (TILE_C * TPAD), TILE_C * TPAD)],
            wsems[b])

    def transpose(b):
        def tbody(bt8, carry):
            btv = jnp.broadcast_to(bt8 * LANES, (LANES,)).astype(jnp.int32)
            btv64 = btv * EMBED_DIM
            for k in range(LANES):
                lcol = btv + rot[k]
                for g in range(GROUPS):
                    v = plsc.load_gather(stage_bufs[b], [ge[g], lcol])
                    plsc.store_scatter(trans_bufs[b],
                                       [btv64 + rot64[k] + ge[g]], v)
            return carry

        lax.fori_loop(0, TILE_C // LANES, tbody, 0, unroll=False)

    for b in range(NBUF - 1):
        @pl.when(col(b) < NUM_TCOLS)
        def _():
            read_copy(b, b).start()

    def outer(o, carry):
        for b in range(NBUF):
            k = o * NBUF + b
            c = col(k)

            @pl.when(c < NUM_TCOLS)
            def _():
                read_copy(b, k).wait()

                @pl.when(k >= NBUF)
                def _():
                    write_copy(b, k - NBUF).wait()

                transpose(b)

            @pl.when(col(k + NBUF - 1) < NUM_TCOLS)
            def _():
                read_copy((b - 1) % NBUF, k + NBUF - 1).start()

            @pl.when(c < NUM_TCOLS)
            def _():
                write_copy(b, k).start()
        return carry

    n_outer = COLS_PER_W // NBUF
    lax.fori_loop(0, n_outer, outer, 0, unroll=False)

    for k in range(n_outer * NBUF, COLS_PER_W):
        b = k % NBUF
        write_copy(b, k - NBUF).wait()

        @pl.when(col(k) < NUM_TCOLS)
        def _():
            read_copy(b, k).wait()
            transpose(b)
            write_copy(b, k).start()

    for k in range(COLS_PER_W - NBUF, COLS_PER_W):
        b = k % NBUF
        if k < n_outer * NBUF:
            write_copy(b, k).wait()
        else:
            @pl.when(col(k) < NUM_TCOLS)
            def _():
                write_copy(b, k).wait()


def _embed_body(idx_hbm, table_hbm, pos_hbm, out_hbm, idx_v, rows_v, out_v,
                pos_v, gsems, wsems):
    wid = _wid()
    b0 = wid * BLOCK_B

    pltpu.sync_copy(pos_hbm, pos_v)
    pltpu.sync_copy(idx_hbm.at[:, pl.ds(b0, BLOCK_B)], idx_v)

    iota = lax.iota(jnp.int32, LANES)
    row_idx = [iota + g * LANES for g in range(GROUPS)]

    def gather_copy(b, l):
        return pltpu.make_async_copy(
            table_hbm.at[idx_v.at[l]], rows_v.at[b], gsems[b])

    def write_copy(b, l):
        return pltpu.make_async_copy(
            out_v.at[b, :, pl.ds(0, BLOCK_B)],
            out_hbm.at[l, :, pl.ds(b0, BLOCK_B)], wsems[b])

    def transpose_add(b, l):
        pos_g = [pos_v[l, pl.ds(g * LANES, LANES)] for g in range(GROUPS)]

        def bbody(bb, carry):
            col_idx = jnp.broadcast_to(bb, (LANES,)).astype(jnp.int32)
            for g in range(GROUPS):
                v = rows_v[b, bb, pl.ds(g * LANES, LANES)] + pos_g[g]
                plsc.store_scatter(out_v.at[b], [row_idx[g], col_idx], v)
            return carry

        lax.fori_loop(0, BLOCK_B, bbody, 0, unroll=4)

    for b in range(NBUF - 1):
        gather_copy(b, b).start()

    def outer(o, carry):
        for b in range(NBUF):
            l = o * NBUF + b
            gather_copy(b, l).wait()

            @pl.when(l >= NBUF)
            def _():
                write_copy(b, l - NBUF).wait()

            transpose_add(b, l)

            @pl.when(l + NBUF - 1 <= SEQ_LEN - 1)
            def _():
                gather_copy((b - 1) % NBUF, l + NBUF - 1).start()

            write_copy(b, l).start()
        return carry

    lax.fori_loop(0, SEQ_LEN // NBUF, outer, 0, unroll=False)

    for b in range(NBUF):
        write_copy(b, SEQ_LEN - NBUF + b).wait()


@jax.jit
def _embed(inputs, token_table, pos_table):
    mesh = plsc.VectorSubcoreMesh(
        core_axis_name="c", subcore_axis_name="s", num_cores=NUM_CORES,
        num_subcores=NUM_SUBCORES)

    repack = pl.kernel(
        _repack_body,
        out_type=jax.ShapeDtypeStruct((TROWS * TPAD,), jnp.float32),
        mesh=mesh,
        scratch_types=[
            [pltpu.VMEM((EMBED_DIM, TILE_C), jnp.float32)] * NBUF,
            [pltpu.VMEM((TILE_C * TPAD,), jnp.float32)] * NBUF,
            [pltpu.SemaphoreType.DMA] * NBUF,
            [pltpu.SemaphoreType.DMA] * NBUF,
        ],
        compiler_params=pltpu.CompilerParams(use_tc_tiling_on_sc=True,
                                             needs_layout_passes=False),
    )
    table_flat = repack(jnp.transpose(token_table))
    table_rm = table_flat.reshape(TROWS, TPAD)

    idx_t = jnp.transpose(inputs)
    f = pl.kernel(
        _embed_body,
        out_type=jax.ShapeDtypeStruct((SEQ_LEN, EMBED_DIM, BATCH),
                                      jnp.float32),
        mesh=mesh,
        scratch_types=[
            pltpu.VMEM((SEQ_LEN, BLOCK_B), jnp.int32),
            pltpu.VMEM((NBUF, BLOCK_B, TPAD), jnp.float32),
            pltpu.VMEM((NBUF, EMBED_DIM, BPAD), jnp.float32),
            pltpu.VMEM((SEQ_LEN, EMBED_DIM), jnp.float32),
            [pltpu.SemaphoreType.DMA] * NBUF,
            [pltpu.SemaphoreType.DMA] * NBUF,
        ],
        compiler_params=pltpu.CompilerParams(use_tc_tiling_on_sc=False,
                                             needs_layout_passes=False),
    )
    out_t = f(idx_t, table_rm, pos_table)
    return jnp.transpose(out_t, (2, 0, 1))


def kernel(inputs, token_table, pos_table):
    return _embed(inputs, token_table, pos_table)


# speedup vs baseline: 2.3263x; 1.3794x over previous
import jax
import jax.numpy as jnp
from jax import lax
from jax.experimental import pallas as pl
from jax.experimental.pallas import tpu as pltpu
from jax.experimental.pallas import tpu_sc as plsc

BATCH = 4096
SEQ_LEN = 200
EMBED_DIM = 64
VOCAB = 1000000
ROW_PAD = 128

NUM_CORES = 2
NUM_SUBCORES = 16
NUM_WORKERS = NUM_CORES * NUM_SUBCORES

BLOCK_B = BATCH // NUM_WORKERS
NBUF = 3
LANES = 16
GROUPS = EMBED_DIM // LANES


def _wid():
    return lax.axis_index("s") * NUM_CORES + lax.axis_index("c")


def _embed_body(idx_hbm, table_hbm, pos_hbm, out_hbm, idx_v, rows_bufs,
                out_bufs, pos_v, gsems, wsems):
    wid = _wid()
    b0 = wid * BLOCK_B

    pltpu.sync_copy(pos_hbm, pos_v)
    pltpu.sync_copy(idx_hbm.at[:, pl.ds(b0, BLOCK_B)], idx_v)

    iota = lax.iota(jnp.int32, LANES)
    rot = [jnp.bitwise_and(iota + k, LANES - 1) for k in range(LANES)]
    ge = [iota + be for be in range(0, EMBED_DIM, LANES)]

    def gather_copy(b, l):
        return pltpu.make_async_copy(
            table_hbm.at[idx_v.at[l]], rows_bufs[b], gsems[b])

    def write_copy(b, l):
        return pltpu.make_async_copy(
            out_bufs[b], out_hbm.at[l, :, pl.ds(b0, BLOCK_B)], wsems[b])

    def transpose_add(b, l):
        pos_g = [pos_v[l, pl.ds(g * LANES, LANES)] for g in range(GROUPS)]

        def tbody(bt8, carry):
            bbv = jnp.broadcast_to(bt8 * LANES, (LANES,)).astype(jnp.int32)
            for k in range(LANES):
                bcol = bbv + rot[k]
                for g in range(GROUPS):
                    v = plsc.load_gather(rows_bufs[b], [bcol, ge[g]])
                    plsc.store_scatter(out_bufs[b], [ge[g], bcol],
                                       v + pos_g[g])
            return carry

        lax.fori_loop(0, BLOCK_B // LANES, tbody, 0, unroll=False)

    for b in range(NBUF - 1):
        gather_copy(b, b).start()

    def outer(o, carry):
        for b in range(NBUF):
            l = o * NBUF + b
            gather_copy(b, l).wait()

            @pl.when(l >= NBUF)
            def _():
                write_copy(b, l - NBUF).wait()

            transpose_add(b, l)

            @pl.when(l + NBUF - 1 <= SEQ_LEN - 1)
            def _():
                gather_copy((b - 1) % NBUF, l + NBUF - 1).start()

            write_copy(b, l).start()
        return carry

    lax.fori_loop(0, SEQ_LEN // NBUF, outer, 0, unroll=False)

    for l in range(SEQ_LEN - SEQ_LEN % NBUF, SEQ_LEN):
        b = l % NBUF
        gather_copy(b, l).wait()
        write_copy(b, l - NBUF).wait()
        transpose_add(b, l)
        write_copy(b, l).start()

    for l in range(SEQ_LEN - NBUF, SEQ_LEN):
        write_copy(l % NBUF, l).wait()


@jax.jit
def _embed(inputs, token_table, pos_table):
    mesh = plsc.VectorSubcoreMesh(
        core_axis_name="c", subcore_axis_name="s", num_cores=NUM_CORES,
        num_subcores=NUM_SUBCORES)

    table_p = jnp.pad(token_table, ((0, 0), (0, ROW_PAD - EMBED_DIM)))
    idx_t = jnp.transpose(inputs)

    f = pl.kernel(
        _embed_body,
        out_type=jax.ShapeDtypeStruct((SEQ_LEN, EMBED_DIM, BATCH),
                                      jnp.float32),
        mesh=mesh,
        scratch_types=[
            pltpu.VMEM((SEQ_LEN, BLOCK_B), jnp.int32),
            [pltpu.VMEM((BLOCK_B, ROW_PAD), jnp.float32)] * NBUF,
            [pltpu.VMEM((EMBED_DIM, BLOCK_B), jnp.float32)] * NBUF,
            pltpu.VMEM((SEQ_LEN, EMBED_DIM), jnp.float32),
            [pltpu.SemaphoreType.DMA] * NBUF,
            [pltpu.SemaphoreType.DMA] * NBUF,
        ],
        compiler_params=pltpu.CompilerParams(use_tc_tiling_on_sc=True,
                                             needs_layout_passes=False),
    )
    out_t = f(idx_t, table_p, pos_table)
    return jnp.transpose(out_t, (2, 0, 1))


def kernel(inputs, token_table, pos_table):
    return _embed(inputs, token_table, pos_table)
